# Initial kernel scaffold; baseline (speedup 1.0000x reference)
import jax
import jax.numpy as jnp
from jax.experimental import pallas as pl


def kernel(data, query, W, b):
    raise NotImplementedError("write your pallas kernel here")


# speedup vs baseline: 1.9274x; 1.9274x over previous
import jax
import jax.numpy as jnp
from jax.experimental import pallas as pl
from jax.experimental.pallas import tpu as pltpu

_BN = 512


def _ranks_t(x_t):
    a = x_t.shape[0]
    eye_inf = jnp.where(
        jax.lax.broadcasted_iota(jnp.int32, (a, a, 1), 0)
        == jax.lax.broadcasted_iota(jnp.int32, (a, a, 1), 1),
        jnp.float32(jnp.inf), jnp.float32(0.0))
    gt = x_t[:, None, :] > (x_t[None, :, :] + eye_inf)
    return 1.0 + jnp.sum(jnp.where(gt, 1.0, 0.0), axis=0)


def _anchor_kernel(data_t_ref, query_t_ref, w_ref, wt_ref, b_col_ref,
                   out_ref, qr_t_ref):
    i = pl.program_id(0)
    w = w_ref[...]
    wt = wt_ref[...]
    b_col = b_col_ref[...]
    norm_col = jnp.sqrt(jnp.sum(wt * wt, axis=1, keepdims=True))

    @pl.when(i == 0)
    def _():
        qx_t = jnp.dot(w, query_t_ref[...],
                       preferred_element_type=jnp.float32) + b_col
        qr_t_ref[...] = _ranks_t(qx_t / norm_col).astype(jnp.bfloat16)

    x_t = jnp.dot(w, data_t_ref[...], preferred_element_type=jnp.float32) + b_col
    r_t = _ranks_t(x_t / norm_col).astype(jnp.bfloat16)
    out_ref[...] = jax.lax.dot_general(
        qr_t_ref[...], r_t, (((0,), (0,)), ((), ())),
        preferred_element_type=jnp.float32)


def kernel(data, query, W, b):
    N, D = data.shape
    Q = query.shape[0]
    A = W.shape[0]
    out = pl.pallas_call(
        _anchor_kernel,
        grid=(N // _BN,),
        in_specs=[
            pl.BlockSpec((D, _BN), lambda i: (0, i)),
            pl.BlockSpec((D, Q), lambda i: (0, 0)),
            pl.BlockSpec((A, D), lambda i: (0, 0)),
            pl.BlockSpec((D, A), lambda i: (0, 0)),
            pl.BlockSpec((A, 1), lambda i: (0, 0)),
        ],
        out_specs=pl.BlockSpec((Q, _BN), lambda i: (0, i)),
        out_shape=jax.ShapeDtypeStruct((Q, N), jnp.float32),
        scratch_shapes=[pltpu.VMEM((A, Q), jnp.bfloat16)],
    )(data.T, query.T, W, W.T, b[:, None])
    return out


# speedup vs baseline: 2.0563x; 1.0669x over previous
import jax
import jax.numpy as jnp
from jax.experimental import pallas as pl
from jax.experimental.pallas import tpu as pltpu

_BN = 512


def _ranks_t(x_t):
    a, b = x_t.shape
    rows = jax.lax.broadcasted_iota(jnp.int32, (a, 1), 0)
    out_chunks = []
    for c in range(0, b, 128):
        x_c = x_t[:, c:c + 128]
        acc = jnp.full((a, 128), 1.0, dtype=jnp.float32)
        for j in range(a):
            inf_col = jnp.where(rows == j, jnp.float32(jnp.inf),
                                jnp.float32(0.0))
            gt = x_c[j:j + 1, :] > (x_c + inf_col)
            acc = acc + jnp.where(gt, 1.0, 0.0)
        out_chunks.append(acc)
    return jnp.concatenate(out_chunks, axis=1)


def _anchor_kernel(data_t_ref, query_t_ref, w_ref, wt_ref, b_col_ref,
                   out_ref, qr_t_ref):
    i = pl.program_id(0)
    w = w_ref[...]
    wt = wt_ref[...]
    b_col = b_col_ref[...]
    norm_col = jnp.sqrt(jnp.sum(wt * wt, axis=1, keepdims=True))

    @pl.when(i == 0)
    def _():
        qx_t = jnp.dot(w, query_t_ref[...],
                       preferred_element_type=jnp.float32) + b_col
        qr_t_ref[...] = _ranks_t(qx_t / norm_col).astype(jnp.bfloat16)

    x_t = jnp.dot(w, data_t_ref[...], preferred_element_type=jnp.float32) + b_col
    r_t = _ranks_t(x_t / norm_col).astype(jnp.bfloat16)
    out_ref[...] = jax.lax.dot_general(
        qr_t_ref[...], r_t, (((0,), (0,)), ((), ())),
        preferred_element_type=jnp.float32)


def kernel(data, query, W, b):
    N, D = data.shape
    Q = query.shape[0]
    A = W.shape[0]
    out = pl.pallas_call(
        _anchor_kernel,
        grid=(N // _BN,),
        in_specs=[
            pl.BlockSpec((D, _BN), lambda i: (0, i)),
            pl.BlockSpec((D, Q), lambda i: (0, 0)),
            pl.BlockSpec((A, D), lambda i: (0, 0)),
            pl.BlockSpec((D, A), lambda i: (0, 0)),
            pl.BlockSpec((A, 1), lambda i: (0, 0)),
        ],
        out_specs=pl.BlockSpec((Q, _BN), lambda i: (0, i)),
        out_shape=jax.ShapeDtypeStruct((Q, N), jnp.float32),
        scratch_shapes=[pltpu.VMEM((A, Q), jnp.bfloat16)],
    )(data.T, query.T, W, W.T, b[:, None])
    return out
